# baseline (speedup 1.0000x reference)
import jax
import jax.numpy as jnp
from jax.experimental import pallas as pl
from jax.experimental.pallas import tpu as pltpu

_N_CHUNKS = 8


def _dma_body(w_hbm, o_hbm, sems):
    n = w_hbm.shape[0]
    rows = n // _N_CHUNKS
    copies = [
        pltpu.make_async_copy(
            w_hbm.at[pl.ds(i * rows, rows), :],
            o_hbm.at[pl.ds(i * rows, rows), :],
            sems.at[i],
        )
        for i in range(_N_CHUNKS)
    ]
    for c in copies:
        c.start()
    for c in copies:
        c.wait()


def kernel(weight, edge_index):
    n, d = weight.shape
    return pl.pallas_call(
        _dma_body,
        out_shape=jax.ShapeDtypeStruct((n, d), weight.dtype),
        in_specs=[pl.BlockSpec(memory_space=pl.ANY)],
        out_specs=pl.BlockSpec(memory_space=pl.ANY),
        scratch_shapes=[pltpu.SemaphoreType.DMA((_N_CHUNKS,))],
    )(weight)


# speedup vs baseline: 30.0199x; 30.0199x over previous
import jax
import jax.numpy as jnp
from jax.experimental import pallas as pl
from jax.experimental.pallas import tpu as pltpu

_BLOCK_ROWS = 2000


def _copy_body(w_ref, o_ref):
    o_ref[...] = w_ref[...]


def kernel(weight, edge_index):
    n, d = weight.shape
    return pl.pallas_call(
        _copy_body,
        out_shape=jax.ShapeDtypeStruct((n, d), weight.dtype),
        grid=(n // _BLOCK_ROWS,),
        in_specs=[pl.BlockSpec((_BLOCK_ROWS, d), lambda i: (i, 0))],
        out_specs=pl.BlockSpec((_BLOCK_ROWS, d), lambda i: (i, 0)),
        compiler_params=pltpu.CompilerParams(
            dimension_semantics=("arbitrary",),
        ),
    )(weight)


# speedup vs baseline: 47.1047x; 1.5691x over previous
import jax
import jax.numpy as jnp
from jax.experimental import pallas as pl
from jax.experimental.pallas import tpu as pltpu

_BLOCK_ROWS = 10000


def _copy_body(w_ref, o_ref):
    o_ref[...] = w_ref[...]


def kernel(weight, edge_index):
    n, d = weight.shape
    return pl.pallas_call(
        _copy_body,
        out_shape=jax.ShapeDtypeStruct((n, d), weight.dtype),
        grid=(n // _BLOCK_ROWS,),
        in_specs=[pl.BlockSpec((_BLOCK_ROWS, d), lambda i: (i, 0))],
        out_specs=pl.BlockSpec((_BLOCK_ROWS, d), lambda i: (i, 0)),
        compiler_params=pltpu.CompilerParams(
            dimension_semantics=("arbitrary",),
        ),
    )(weight)


# speedup vs baseline: 49.2838x; 1.0463x over previous
import jax
import jax.numpy as jnp
from jax.experimental import pallas as pl
from jax.experimental.pallas import tpu as pltpu

_BLOCK_ROWS = 20000


def _copy_body(w_ref, o_ref):
    o_ref[...] = w_ref[...]


def kernel(weight, edge_index):
    n, d = weight.shape
    return pl.pallas_call(
        _copy_body,
        out_shape=jax.ShapeDtypeStruct((n, d), weight.dtype),
        grid=(n // _BLOCK_ROWS,),
        in_specs=[pl.BlockSpec((_BLOCK_ROWS, d), lambda i: (i, 0))],
        out_specs=pl.BlockSpec((_BLOCK_ROWS, d), lambda i: (i, 0)),
        compiler_params=pltpu.CompilerParams(
            dimension_semantics=("arbitrary",),
        ),
    )(weight)
